# baseline (speedup 1.0000x reference)
import functools

import jax
import jax.numpy as jnp
from jax import lax
from jax.experimental import pallas as pl
from jax.experimental.pallas import tpu as pltpu
from jax.experimental.pallas import tpu_sc as plsc

N_NODES = 10000
D_IN = 128
D_HID = 256
D_OUT = 128
QW = 32
NEP = D_IN // QW

NC = 2
NS = 16
NW = NC * NS
CH = 256
NCHUNK = 40
E_PAD = NW * NCHUNK * CH
N_PAD = 10112
ROWS_PER_TILE = N_PAD // NS
XROWS_PER_TILE = N_NODES // NS
DEG_W = 16
NB = 4
GB = 2


@functools.lru_cache(maxsize=None)
def _sc_segment_sum(with_deg: bool):
  out_type = [jax.ShapeDtypeStruct((NC, N_PAD, D_IN), jnp.float32)]
  scratch = [
      pltpu.VMEM((NCHUNK, CH), jnp.int32),
      pltpu.VMEM((NCHUNK, CH), jnp.int32),
      pltpu.VMEM((NB, CH, QW), jnp.float32),
      pltpu.VMEM((CH, QW), jnp.float32),
      pltpu.VMEM_SHARED((N_NODES, QW), jnp.float32),
      pltpu.VMEM_SHARED((N_PAD, QW), jnp.float32),
      [pltpu.SemaphoreType.DMA] * NB,
      pltpu.SemaphoreType.DMA,
      pltpu.SemaphoreType.DMA,
  ]
  if with_deg:
    out_type.append(jax.ShapeDtypeStruct((NC, N_PAD, DEG_W), jnp.float32))
    scratch += [
        pltpu.VMEM((CH, DEG_W), jnp.float32),
        pltpu.VMEM_SHARED((N_PAD, DEG_W), jnp.float32),
    ]

  mesh = plsc.VectorSubcoreMesh(core_axis_name="c", subcore_axis_name="s",
                                num_cores=NC, num_subcores=NS)

  def body(feat_hbm, src_hbm, dst_hbm,
           acc_out, deg_out, idx_s, idx_d, rows, zbuf, x_sh, acc_sh,
           gsem, ssem, dsem, ones, deg_sh):
    c = lax.axis_index("c")
    s = lax.axis_index("s")
    wid = s * NC + c
    r0 = s * ROWS_PER_TILE
    x0 = s * XROWS_PER_TILE
    zero16 = jnp.zeros((16,), jnp.float32)

    pltpu.sync_copy(src_hbm.at[wid], idx_s)
    pltpu.sync_copy(dst_hbm.at[wid], idx_d)

    def zero_zbuf(i, _):
      for j in range(QW // 16):
        zbuf[i, pl.ds(j * 16, 16)] = zero16
      return 0
    lax.fori_loop(0, CH, zero_zbuf, 0)

    if with_deg:
      one16 = jnp.where(lax.iota(jnp.int32, 16) == 0,
                        jnp.float32(1.0), jnp.float32(0.0))

      def init_ones(i, _):
        ones[i] = zero16
        return 0
      lax.fori_loop(0, CH, init_ones, 0)
      n_fulld = ROWS_PER_TILE // CH
      for q in range(n_fulld):
        pltpu.sync_copy(ones, deg_sh.at[pl.ds(r0 + q * CH, CH)])
      remd = ROWS_PER_TILE - n_fulld * CH
      if remd:
        pltpu.sync_copy(ones.at[pl.ds(0, remd)],
                        deg_sh.at[pl.ds(r0 + n_fulld * CH, remd)])

      def init_ones2(i, _):
        ones[i] = one16
        return 0
      lax.fori_loop(0, CH, init_ones2, 0)

    for epoch in range(NEP):
      deg_here = with_deg and epoch == 0

      pltpu.sync_copy(
          feat_hbm.at[pl.ds(x0, XROWS_PER_TILE), pl.ds(epoch * QW, QW)],
          x_sh.at[pl.ds(x0, XROWS_PER_TILE)])
      n_full = ROWS_PER_TILE // CH
      for q in range(n_full):
        pltpu.sync_copy(zbuf, acc_sh.at[pl.ds(r0 + q * CH, CH)])
      rem = ROWS_PER_TILE - n_full * CH
      if rem:
        pltpu.sync_copy(zbuf.at[pl.ds(0, rem)],
                        acc_sh.at[pl.ds(r0 + n_full * CH, rem)])
      plsc.subcore_barrier()

      ngroup = NCHUNK // GB

      def g_start_group(g, base):
        for i in range(GB):
          pltpu.async_copy(x_sh.at[idx_s.at[g * GB + i]],
                           rows.at[base + i], gsem[base + i])

      def handle_group(g, base, have_next):
        other = 0 if base else GB
        if deg_here:
          for i in range(GB):
            pltpu.async_copy(ones, deg_sh.at[idx_d.at[g * GB + i]], dsem,
                             add=True)

        @pl.when(have_next)
        def _():
          g_start_group(g + 1, other)
        for i in range(GB):
          pltpu.make_async_copy(x_sh.at[idx_s.at[0]],
                                rows.at[base + i], gsem[base + i]).wait()
          pltpu.async_copy(rows.at[base + i],
                           acc_sh.at[idx_d.at[g * GB + i]], ssem, add=True)
        for i in range(GB):
          pltpu.make_async_copy(rows.at[base + i],
                                acc_sh.at[idx_d.at[0]], ssem).wait()
        if deg_here:
          for i in range(GB):
            pltpu.make_async_copy(ones, deg_sh.at[idx_d.at[0]],
                                  dsem).wait()

      g_start_group(0, 0)

      def step(p, _):
        handle_group(2 * p, 0, jnp.bool_(True))
        handle_group(2 * p + 1, GB, 2 * p + 2 < ngroup)
        return 0
      lax.fori_loop(0, ngroup // 2, step, 0)
      plsc.subcore_barrier()

      pltpu.sync_copy(
          acc_sh.at[pl.ds(r0, ROWS_PER_TILE)],
          acc_out.at[c, pl.ds(r0, ROWS_PER_TILE), pl.ds(epoch * QW, QW)])
      if deg_here:
        pltpu.sync_copy(deg_sh.at[pl.ds(r0, ROWS_PER_TILE)],
                        deg_out.at[c, pl.ds(r0, ROWS_PER_TILE)])

  if with_deg:
    def fn(feat_hbm, src_hbm, dst_hbm,
           acc_out, deg_out, idx_s, idx_d, rows, zbuf, x_sh, acc_sh,
           gsem, ssem, dsem, ones, deg_sh):
      body(feat_hbm, src_hbm, dst_hbm,
           acc_out, deg_out, idx_s, idx_d, rows, zbuf, x_sh, acc_sh,
           gsem, ssem, dsem, ones, deg_sh)
  else:
    def fn(feat_hbm, src_hbm, dst_hbm,
           acc_out, idx_s, idx_d, rows, zbuf, x_sh, acc_sh,
           gsem, ssem, dsem):
      body(feat_hbm, src_hbm, dst_hbm,
           acc_out, None, idx_s, idx_d, rows, zbuf, x_sh, acc_sh,
           gsem, ssem, dsem, None, None)

  return pl.kernel(fn, out_type=tuple(out_type), mesh=mesh,
                   scratch_types=scratch,
                   compiler_params=pltpu.CompilerParams(
                       use_tc_tiling_on_sc=False))


_BLK = 1000


def _combine_acc(acc, deg):
  d = deg[0, :, 0:1] + deg[1, :, 0:1]
  scale = 1.0 / jnp.maximum(d, 1.0)
  return (acc[0] + acc[1]) * scale


def _tc_xr_body(x_ref, wr1_ref, b1_ref, xr_ref):
  xr_ref[...] = jnp.dot(x_ref[...], wr1_ref[...],
                        preferred_element_type=jnp.float32) + b1_ref[...]


def _tc_xr(x, W_r1, b1):
  grid = (N_NODES // _BLK,)
  return pl.pallas_call(
      _tc_xr_body,
      grid=grid,
      in_specs=[
          pl.BlockSpec((_BLK, D_IN), lambda i: (i, 0)),
          pl.BlockSpec((D_IN, D_HID), lambda i: (0, 0)),
          pl.BlockSpec((1, D_HID), lambda i: (0, 0)),
      ],
      out_specs=pl.BlockSpec((_BLK, D_HID), lambda i: (i, 0)),
      out_shape=jax.ShapeDtypeStruct((N_NODES, D_HID), jnp.float32),
  )(x, W_r1, b1)


def _tc_layer1_body(xr_ref, acc_ref, deg_ref, wl1_ref,
                    wl2_ref, wr2_ref, b2_ref, z_ref, r_ref):
  agg = _combine_acc(acc_ref[...], deg_ref[...])
  h = jnp.dot(agg, wl1_ref[...], preferred_element_type=jnp.float32)
  h += xr_ref[...]
  h = jnp.maximum(h, 0.0)
  z_ref[...] = jnp.dot(h, wl2_ref[...], preferred_element_type=jnp.float32)
  r_ref[...] = jnp.dot(h, wr2_ref[...],
                       preferred_element_type=jnp.float32) + b2_ref[...]


def _tc_layer1(xr, acc, deg, W_l1, W_l2, W_r2, b2):
  grid = (N_NODES // _BLK,)
  return pl.pallas_call(
      _tc_layer1_body,
      grid=grid,
      in_specs=[
          pl.BlockSpec((_BLK, D_HID), lambda i: (i, 0)),
          pl.BlockSpec((NC, _BLK, D_IN), lambda i: (0, i, 0)),
          pl.BlockSpec((NC, _BLK, DEG_W), lambda i: (0, i, 0)),
          pl.BlockSpec((D_IN, D_HID), lambda i: (0, 0)),
          pl.BlockSpec((D_HID, D_OUT), lambda i: (0, 0)),
          pl.BlockSpec((D_HID, D_OUT), lambda i: (0, 0)),
          pl.BlockSpec((1, D_OUT), lambda i: (0, 0)),
      ],
      out_specs=[
          pl.BlockSpec((_BLK, D_OUT), lambda i: (i, 0)),
          pl.BlockSpec((_BLK, D_OUT), lambda i: (i, 0)),
      ],
      out_shape=[
          jax.ShapeDtypeStruct((N_NODES, D_OUT), jnp.float32),
          jax.ShapeDtypeStruct((N_NODES, D_OUT), jnp.float32),
      ],
  )(xr, acc, deg, W_l1, W_l2, W_r2, b2)


def _tc_final_body(acc_ref, deg_ref, r_ref, out_ref):
  out_ref[...] = _combine_acc(acc_ref[...], deg_ref[...]) + r_ref[...]


def _tc_final(acc, deg, r):
  grid = (N_NODES // _BLK,)
  return pl.pallas_call(
      _tc_final_body,
      grid=grid,
      in_specs=[
          pl.BlockSpec((NC, _BLK, D_OUT), lambda i: (0, i, 0)),
          pl.BlockSpec((NC, _BLK, DEG_W), lambda i: (0, i, 0)),
          pl.BlockSpec((_BLK, D_OUT), lambda i: (i, 0)),
      ],
      out_specs=pl.BlockSpec((_BLK, D_OUT), lambda i: (i, 0)),
      out_shape=jax.ShapeDtypeStruct((N_NODES, D_OUT), jnp.float32),
  )(acc, deg, r)


@jax.jit
def kernel(x, edge_index, W_l1, W_r1, b1, W_l2, W_r2, b2):
  src = edge_index[0].astype(jnp.int32)
  dst = edge_index[1].astype(jnp.int32)
  n_edges = src.shape[0]
  pad = E_PAD - n_edges
  pad_ar = jnp.arange(pad, dtype=jnp.int32)
  src_p = jnp.concatenate([src, pad_ar % N_NODES])
  dst_p = jnp.concatenate([dst, N_NODES + pad_ar % (N_PAD - N_NODES)])
  src_p = src_p.reshape(NW, NCHUNK, CH)
  dst_p = dst_p.reshape(NW, NCHUNK, CH)

  b1r = b1.reshape(1, D_HID)
  b2r = b2.reshape(1, D_OUT)
  xr = _tc_xr(x, W_r1, b1r)
  acc1, degw = _sc_segment_sum(True)(x, src_p, dst_p)
  z, r = _tc_layer1(xr, acc1, degw, W_l1, W_l2, W_r2, b2r)
  (acc2,) = _sc_segment_sum(False)(z, src_p, dst_p)
  return _tc_final(acc2, degw, r)


# speedup vs baseline: 1.0354x; 1.0354x over previous
import functools

import jax
import jax.numpy as jnp
from jax import lax
from jax.experimental import pallas as pl
from jax.experimental.pallas import tpu as pltpu
from jax.experimental.pallas import tpu_sc as plsc

N_NODES = 10000
D_IN = 128
D_HID = 256
D_OUT = 128
QW = 32
NEP = D_IN // QW

NC = 2
NS = 16
NW = NC * NS
CH = 128
NCHUNK = 80
E_PAD = NW * NCHUNK * CH
N_PAD = 10112
ROWS_PER_TILE = N_PAD // NS
XROWS_PER_TILE = N_NODES // NS
DEG_W = 16
NB = 8
GB = 4


@functools.lru_cache(maxsize=None)
def _sc_segment_sum(with_deg: bool):
  out_type = [jax.ShapeDtypeStruct((NC, N_PAD, D_IN), jnp.float32)]
  scratch = [
      pltpu.VMEM((NCHUNK, CH), jnp.int32),
      pltpu.VMEM((NCHUNK, CH), jnp.int32),
      pltpu.VMEM((NB, CH, QW), jnp.float32),
      pltpu.VMEM_SHARED((2, N_NODES, QW), jnp.float32),
      pltpu.VMEM_SHARED((N_PAD, QW), jnp.float32),
      [pltpu.SemaphoreType.DMA] * NB,
      pltpu.SemaphoreType.DMA,
      pltpu.SemaphoreType.DMA,
      pltpu.SemaphoreType.DMA,
  ]
  if with_deg:
    out_type.append(jax.ShapeDtypeStruct((NC, N_PAD, DEG_W), jnp.float32))
    scratch += [
        pltpu.VMEM((CH, DEG_W), jnp.float32),
        pltpu.VMEM_SHARED((N_PAD, DEG_W), jnp.float32),
    ]

  mesh = plsc.VectorSubcoreMesh(core_axis_name="c", subcore_axis_name="s",
                                num_cores=NC, num_subcores=NS)

  def body(feat_hbm, src_hbm, dst_hbm,
           acc_out, deg_out, idx_s, idx_d, rows, x_sh, acc_sh,
           gsem, ssem, dsem, xsem, ones, deg_sh):
    c = lax.axis_index("c")
    s = lax.axis_index("s")
    wid = s * NC + c
    r0 = s * ROWS_PER_TILE
    x0 = s * XROWS_PER_TILE
    zero16 = jnp.zeros((16,), jnp.float32)

    def stage_quarter(epoch):
      pltpu.async_copy(
          feat_hbm.at[pl.ds(x0, XROWS_PER_TILE), pl.ds(epoch * QW, QW)],
          x_sh.at[epoch % 2, pl.ds(x0, XROWS_PER_TILE)], xsem)

    def stage_wait():
      pltpu.make_async_copy(
          feat_hbm.at[pl.ds(x0, XROWS_PER_TILE), pl.ds(0, QW)],
          x_sh.at[0, pl.ds(x0, XROWS_PER_TILE)], xsem).wait()

    stage_quarter(0)
    pltpu.sync_copy(src_hbm.at[wid], idx_s)
    pltpu.sync_copy(dst_hbm.at[wid], idx_d)

    if with_deg:
      one16 = jnp.where(lax.iota(jnp.int32, 16) == 0,
                        jnp.float32(1.0), jnp.float32(0.0))

      def init_ones(i, _):
        ones[i] = zero16
        return 0
      lax.fori_loop(0, CH, init_ones, 0)
      n_fulld = ROWS_PER_TILE // CH
      for q in range(n_fulld):
        pltpu.sync_copy(ones, deg_sh.at[pl.ds(r0 + q * CH, CH)])
      remd = ROWS_PER_TILE - n_fulld * CH
      if remd:
        pltpu.sync_copy(ones.at[pl.ds(0, remd)],
                        deg_sh.at[pl.ds(r0 + n_fulld * CH, remd)])

      def init_ones2(i, _):
        ones[i] = one16
        return 0
      lax.fori_loop(0, CH, init_ones2, 0)

    for epoch in range(NEP):
      deg_here = with_deg and epoch == 0

      if epoch + 1 < NEP:
        stage_quarter(epoch + 1)

      def zero_buf(i, _):
        for j in range(QW // 16):
          rows[0, i, pl.ds(j * 16, 16)] = zero16
        return 0
      lax.fori_loop(0, CH, zero_buf, 0)
      n_full = ROWS_PER_TILE // CH
      for q in range(n_full):
        pltpu.sync_copy(rows.at[0], acc_sh.at[pl.ds(r0 + q * CH, CH)])
      rem = ROWS_PER_TILE - n_full * CH
      if rem:
        pltpu.sync_copy(rows.at[0, pl.ds(0, rem)],
                        acc_sh.at[pl.ds(r0 + n_full * CH, rem)])
      stage_wait()
      plsc.subcore_barrier()

      ngroup = NCHUNK // GB

      x_cur = x_sh.at[epoch % 2]

      def g_start_group(g, base):
        for i in range(GB):
          pltpu.async_copy(x_cur.at[idx_s.at[g * GB + i]],
                           rows.at[base + i], gsem[base + i])

      def handle_group(g, base, have_next):
        other = 0 if base else GB
        if deg_here:
          for i in range(GB):
            pltpu.async_copy(ones, deg_sh.at[idx_d.at[g * GB + i]], dsem,
                             add=True)

        @pl.when(have_next)
        def _():
          g_start_group(g + 1, other)
        for i in range(GB):
          pltpu.make_async_copy(x_cur.at[idx_s.at[0]],
                                rows.at[base + i], gsem[base + i]).wait()
          pltpu.async_copy(rows.at[base + i],
                           acc_sh.at[idx_d.at[g * GB + i]], ssem, add=True)
        for i in range(GB):
          pltpu.make_async_copy(rows.at[base + i],
                                acc_sh.at[idx_d.at[0]], ssem).wait()
        if deg_here:
          for i in range(GB):
            pltpu.make_async_copy(ones, deg_sh.at[idx_d.at[0]],
                                  dsem).wait()

      g_start_group(0, 0)

      def step(p, _):
        handle_group(2 * p, 0, jnp.bool_(True))
        handle_group(2 * p + 1, GB, 2 * p + 2 < ngroup)
        return 0
      lax.fori_loop(0, ngroup // 2, step, 0)
      plsc.subcore_barrier()

      pltpu.sync_copy(
          acc_sh.at[pl.ds(r0, ROWS_PER_TILE)],
          acc_out.at[c, pl.ds(r0, ROWS_PER_TILE), pl.ds(epoch * QW, QW)])
      if deg_here:
        pltpu.sync_copy(deg_sh.at[pl.ds(r0, ROWS_PER_TILE)],
                        deg_out.at[c, pl.ds(r0, ROWS_PER_TILE)])

  if with_deg:
    def fn(feat_hbm, src_hbm, dst_hbm,
           acc_out, deg_out, idx_s, idx_d, rows, x_sh, acc_sh,
           gsem, ssem, dsem, xsem, ones, deg_sh):
      body(feat_hbm, src_hbm, dst_hbm,
           acc_out, deg_out, idx_s, idx_d, rows, x_sh, acc_sh,
           gsem, ssem, dsem, xsem, ones, deg_sh)
  else:
    def fn(feat_hbm, src_hbm, dst_hbm,
           acc_out, idx_s, idx_d, rows, x_sh, acc_sh,
           gsem, ssem, dsem, xsem):
      body(feat_hbm, src_hbm, dst_hbm,
           acc_out, None, idx_s, idx_d, rows, x_sh, acc_sh,
           gsem, ssem, dsem, xsem, None, None)

  return pl.kernel(fn, out_type=tuple(out_type), mesh=mesh,
                   scratch_types=scratch,
                   compiler_params=pltpu.CompilerParams(
                       use_tc_tiling_on_sc=False))


_BLK = 1000


def _combine_acc(acc, deg):
  d = deg[0, :, 0:1] + deg[1, :, 0:1]
  scale = 1.0 / jnp.maximum(d, 1.0)
  return (acc[0] + acc[1]) * scale


def _tc_xr_body(x_ref, wr1_ref, b1_ref, xr_ref):
  xr_ref[...] = jnp.dot(x_ref[...], wr1_ref[...],
                        preferred_element_type=jnp.float32) + b1_ref[...]


def _tc_xr(x, W_r1, b1):
  grid = (N_NODES // _BLK,)
  return pl.pallas_call(
      _tc_xr_body,
      grid=grid,
      in_specs=[
          pl.BlockSpec((_BLK, D_IN), lambda i: (i, 0)),
          pl.BlockSpec((D_IN, D_HID), lambda i: (0, 0)),
          pl.BlockSpec((1, D_HID), lambda i: (0, 0)),
      ],
      out_specs=pl.BlockSpec((_BLK, D_HID), lambda i: (i, 0)),
      out_shape=jax.ShapeDtypeStruct((N_NODES, D_HID), jnp.float32),
  )(x, W_r1, b1)


def _tc_layer1_body(xr_ref, acc_ref, deg_ref, wl1_ref,
                    wl2_ref, wr2_ref, b2_ref, z_ref, r_ref):
  agg = _combine_acc(acc_ref[...], deg_ref[...])
  h = jnp.dot(agg, wl1_ref[...], preferred_element_type=jnp.float32)
  h += xr_ref[...]
  h = jnp.maximum(h, 0.0)
  z_ref[...] = jnp.dot(h, wl2_ref[...], preferred_element_type=jnp.float32)
  r_ref[...] = jnp.dot(h, wr2_ref[...],
                       preferred_element_type=jnp.float32) + b2_ref[...]


def _tc_layer1(xr, acc, deg, W_l1, W_l2, W_r2, b2):
  grid = (N_NODES // _BLK,)
  return pl.pallas_call(
      _tc_layer1_body,
      grid=grid,
      in_specs=[
          pl.BlockSpec((_BLK, D_HID), lambda i: (i, 0)),
          pl.BlockSpec((NC, _BLK, D_IN), lambda i: (0, i, 0)),
          pl.BlockSpec((NC, _BLK, DEG_W), lambda i: (0, i, 0)),
          pl.BlockSpec((D_IN, D_HID), lambda i: (0, 0)),
          pl.BlockSpec((D_HID, D_OUT), lambda i: (0, 0)),
          pl.BlockSpec((D_HID, D_OUT), lambda i: (0, 0)),
          pl.BlockSpec((1, D_OUT), lambda i: (0, 0)),
      ],
      out_specs=[
          pl.BlockSpec((_BLK, D_OUT), lambda i: (i, 0)),
          pl.BlockSpec((_BLK, D_OUT), lambda i: (i, 0)),
      ],
      out_shape=[
          jax.ShapeDtypeStruct((N_NODES, D_OUT), jnp.float32),
          jax.ShapeDtypeStruct((N_NODES, D_OUT), jnp.float32),
      ],
  )(xr, acc, deg, W_l1, W_l2, W_r2, b2)


def _tc_final_body(acc_ref, deg_ref, r_ref, out_ref):
  out_ref[...] = _combine_acc(acc_ref[...], deg_ref[...]) + r_ref[...]


def _tc_final(acc, deg, r):
  grid = (N_NODES // _BLK,)
  return pl.pallas_call(
      _tc_final_body,
      grid=grid,
      in_specs=[
          pl.BlockSpec((NC, _BLK, D_OUT), lambda i: (0, i, 0)),
          pl.BlockSpec((NC, _BLK, DEG_W), lambda i: (0, i, 0)),
          pl.BlockSpec((_BLK, D_OUT), lambda i: (i, 0)),
      ],
      out_specs=pl.BlockSpec((_BLK, D_OUT), lambda i: (i, 0)),
      out_shape=jax.ShapeDtypeStruct((N_NODES, D_OUT), jnp.float32),
  )(acc, deg, r)


@jax.jit
def kernel(x, edge_index, W_l1, W_r1, b1, W_l2, W_r2, b2):
  src = edge_index[0].astype(jnp.int32)
  dst = edge_index[1].astype(jnp.int32)
  n_edges = src.shape[0]
  pad = E_PAD - n_edges
  pad_ar = jnp.arange(pad, dtype=jnp.int32)
  src_p = jnp.concatenate([src, pad_ar % N_NODES])
  dst_p = jnp.concatenate([dst, N_NODES + pad_ar % (N_PAD - N_NODES)])
  src_p = src_p.reshape(NW, NCHUNK, CH)
  dst_p = dst_p.reshape(NW, NCHUNK, CH)

  b1r = b1.reshape(1, D_HID)
  b2r = b2.reshape(1, D_OUT)
  xr = _tc_xr(x, W_r1, b1r)
  acc1, degw = _sc_segment_sum(True)(x, src_p, dst_p)
  z, r = _tc_layer1(xr, acc1, degw, W_l1, W_l2, W_r2, b2r)
  (acc2,) = _sc_segment_sum(False)(z, src_p, dst_p)
  return _tc_final(acc2, degw, r)
